# Initial kernel scaffold; baseline (speedup 1.0000x reference)
import jax
import jax.numpy as jnp
from jax.experimental import pallas as pl


def kernel(x, edge_index, W1, b1, W2, b2, W_out, b_out):
    raise NotImplementedError("write your pallas kernel here")


# speedup vs baseline: 9.9706x; 9.9706x over previous
import functools

import jax
import jax.numpy as jnp
from jax import lax
from jax.experimental import pallas as pl
from jax.experimental.pallas import tpu as pltpu
from jax.experimental.pallas import tpu_sc as plsc

NC = 2
NS = 16
NW = NC * NS
CHUNK = 128
DEG_W = 16


def _sc_mesh():
    return plsc.VectorSubcoreMesh(
        core_axis_name="c", subcore_axis_name="s", num_cores=NC, num_subcores=NS
    )


def _make_deg_kernel(n_pad, g_chunks, rpt):
    @functools.partial(
        pl.kernel,
        out_type=jax.ShapeDtypeStruct((NC, n_pad, DEG_W), jnp.float32),
        mesh=_sc_mesh(),
        scratch_types=[
            pltpu.VMEM((g_chunks, CHUNK), jnp.int32),
            pltpu.VMEM((CHUNK, DEG_W), jnp.float32),
            pltpu.VMEM_SHARED((n_pad, DEG_W), jnp.float32),
        ],
    )
    def deg_kernel(dst_hbm, ones_hbm, zeros_hbm, out_hbm, dst_t, ones_t, acc):
        cid = lax.axis_index("c")
        sid = lax.axis_index("s")
        wid = sid * NC + cid

        pltpu.sync_copy(dst_hbm.at[wid], dst_t)
        pltpu.sync_copy(ones_hbm, ones_t)
        pltpu.sync_copy(zeros_hbm, acc.at[pl.ds(sid * rpt, rpt)])
        plsc.subcore_barrier()

        def step(g, carry):
            pltpu.sync_copy(ones_t, acc.at[dst_t.at[g]], add=True)
            return carry

        lax.fori_loop(0, g_chunks, step, 0)
        plsc.subcore_barrier()
        pltpu.sync_copy(
            acc.at[pl.ds(sid * rpt, rpt)],
            out_hbm.at[cid, pl.ds(sid * rpt, rpt)],
        )

    return deg_kernel


def _make_agg_kernel(n, d, n_pad, g_chunks, rpt):
    @functools.partial(
        pl.kernel,
        out_type=jax.ShapeDtypeStruct((NC, n_pad, d), jnp.float32),
        mesh=_sc_mesh(),
        scratch_types=[
            pltpu.VMEM((g_chunks, CHUNK), jnp.int32),
            pltpu.VMEM((g_chunks, CHUNK), jnp.int32),
            pltpu.VMEM((CHUNK, d), jnp.float32),
            pltpu.VMEM_SHARED((n_pad, d), jnp.float32),
            pltpu.SemaphoreType.DMA,
        ],
    )
    def agg_kernel(h_hbm, src_hbm, dst_hbm, zeros_hbm, out_hbm,
                   src_t, dst_t, rows, acc, sem):
        cid = lax.axis_index("c")
        sid = lax.axis_index("s")
        wid = sid * NC + cid

        pltpu.sync_copy(src_hbm.at[wid], src_t)
        pltpu.sync_copy(dst_hbm.at[wid], dst_t)
        pltpu.sync_copy(zeros_hbm, acc.at[pl.ds(sid * rpt, rpt)])
        plsc.subcore_barrier()

        def step(g, carry):
            pltpu.async_copy(h_hbm.at[src_t.at[g]], rows, sem).wait()
            pltpu.sync_copy(rows, acc.at[dst_t.at[g]], add=True)
            return carry

        lax.fori_loop(0, g_chunks, step, 0)
        plsc.subcore_barrier()
        pltpu.sync_copy(
            acc.at[pl.ds(sid * rpt, rpt)],
            out_hbm.at[cid, pl.ds(sid * rpt, rpt)],
        )

    return agg_kernel


def _dis(degp_ref):
    deg = degp_ref[0, :, 0:1] + degp_ref[1, :, 0:1] + 1.0
    return lax.rsqrt(deg)


def _mm_pre_body(x_ref, w_ref, degp_ref, o_ref):
    h = jnp.dot(x_ref[...], w_ref[...], preferred_element_type=jnp.float32)
    o_ref[...] = h * _dis(degp_ref)


def _mm_mid_body(accp_ref, h_ref, degp_ref, b_ref, w_ref, o_ref):
    dis = _dis(degp_ref)
    s = (accp_ref[0] + accp_ref[1] + h_ref[...]) * dis + b_ref[...]
    a = jnp.maximum(s, 0.0)
    o_ref[...] = jnp.dot(a, w_ref[...], preferred_element_type=jnp.float32) * dis


def _final_body(d_out, accp_ref, h_ref, degp_ref, b_ref, o_ref):
    dis = _dis(degp_ref)
    s = (accp_ref[0] + accp_ref[1] + h_ref[...]) * dis + b_ref[...]
    o_ref[...] = s[:, :d_out]


def _row_block(n):
    for r in (2000, 1000, 500, 250):
        if n % r == 0:
            return r
    return n


def kernel(x, edge_index, W1, b1, W2, b2, W_out, b_out):
    n, d_in = x.shape
    e = edge_index.shape[1]
    d_hid = W1.shape[1]
    d_out = W_out.shape[1]
    f32 = jnp.float32

    g_chunks = -(-e // (NW * CHUNK))
    e_pad = NW * CHUNK * g_chunks
    pad = e_pad - e
    src_p = jnp.concatenate(
        [edge_index[0], jnp.zeros((pad,), jnp.int32)]).reshape(NW, g_chunks, CHUNK)
    dst_p = jnp.concatenate(
        [edge_index[1], jnp.full((pad,), n, jnp.int32)]).reshape(NW, g_chunks, CHUNK)

    rpt = -(-(n + 8) // (NS * 8)) * 8
    n_pad = rpt * NS

    ones = jnp.ones((CHUNK, DEG_W), f32)
    zeros_deg = jnp.zeros((rpt, DEG_W), f32)
    zeros_hid = jnp.zeros((rpt, d_hid), f32)
    W_out_p = jnp.pad(W_out, ((0, 0), (0, d_hid - d_out)))
    b_out_p = jnp.pad(b_out, (0, d_hid - d_out))

    deg_kernel = _make_deg_kernel(n_pad, g_chunks, rpt)
    agg_hid = _make_agg_kernel(n, d_hid, n_pad, g_chunks, rpt)

    r = _row_block(n)
    grid = (n // r,)

    def degp_spec():
        return pl.BlockSpec((NC, r, DEG_W), lambda i: (0, i, 0))

    def accp_spec(d):
        return pl.BlockSpec((NC, r, d), lambda i: (0, i, 0))

    mm_pre = pl.pallas_call(
        _mm_pre_body,
        grid=grid,
        in_specs=[
            pl.BlockSpec((r, d_in), lambda i: (i, 0)),
            pl.BlockSpec((d_in, d_hid), lambda i: (0, 0)),
            degp_spec(),
        ],
        out_specs=pl.BlockSpec((r, d_hid), lambda i: (i, 0)),
        out_shape=jax.ShapeDtypeStruct((n, d_hid), f32),
    )

    def mm_mid(d_next):
        return pl.pallas_call(
            _mm_mid_body,
            grid=grid,
            in_specs=[
                accp_spec(d_hid),
                pl.BlockSpec((r, d_hid), lambda i: (i, 0)),
                degp_spec(),
                pl.BlockSpec((1, d_hid), lambda i: (0, 0)),
                pl.BlockSpec((d_hid, d_next), lambda i: (0, 0)),
            ],
            out_specs=pl.BlockSpec((r, d_next), lambda i: (i, 0)),
            out_shape=jax.ShapeDtypeStruct((n, d_next), f32),
        )

    final = pl.pallas_call(
        functools.partial(_final_body, d_out),
        grid=grid,
        in_specs=[
            accp_spec(d_hid),
            pl.BlockSpec((r, d_hid), lambda i: (i, 0)),
            degp_spec(),
            pl.BlockSpec((1, d_hid), lambda i: (0, 0)),
        ],
        out_specs=pl.BlockSpec((r, d_out), lambda i: (i, 0)),
        out_shape=jax.ShapeDtypeStruct((n, d_out), f32),
    )

    degp = deg_kernel(dst_p, ones, zeros_deg)
    h1 = mm_pre(x, W1, degp)
    a1 = agg_hid(h1, src_p, dst_p, zeros_hid)
    h2 = mm_mid(d_hid)(a1, h1, degp, b1.reshape(1, -1), W2)
    a2 = agg_hid(h2, src_p, dst_p, zeros_hid)
    h3 = mm_mid(d_hid)(a2, h2, degp, b2.reshape(1, -1), W_out_p)
    a3 = agg_hid(h3, src_p, dst_p, zeros_hid)
    out = final(a3, h3, degp, b_out_p.reshape(1, -1))
    return out
